# baseline (speedup 1.0000x reference)
import functools

import jax
import jax.numpy as jnp
from jax import lax
from jax.experimental import pallas as pl
from jax.experimental.pallas import tpu as pltpu
from jax.experimental.pallas import tpu_sc as plsc

VOCAB = 100001
EMBED_DIM = 32
DENSE_OUT = 32
BATCH = 16384

_WIDE = 32768
_ABLK = 4096
_AGRID = _WIDE // _ABLK
_LANE_BLOCKS = VOCAB // _ABLK


def _project_pack_tc(table_t, W4, b128):
    def body(t0, t1, t2, t3, w_ref, b_ref, o_ref):
        x4 = jnp.concatenate([t0[...], t1[...], t2[...], t3[...]], axis=0)
        o_ref[...] = (
            lax.dot_general(
                x4,
                w_ref[...],
                (((0,), (0,)), ((), ())),
                preferred_element_type=jnp.float32,
            )
            + b_ref[...]
        )

    def t_spec(j):
        return pl.BlockSpec(
            (EMBED_DIM, _ABLK),
            lambda i: (0, jnp.minimum(i + j * _AGRID, _LANE_BLOCKS)),
        )

    return pl.pallas_call(
        body,
        grid=(_AGRID,),
        in_specs=[
            t_spec(0),
            t_spec(1),
            t_spec(2),
            t_spec(3),
            pl.BlockSpec((128, 128), lambda i: (0, 0)),
            pl.BlockSpec((1, 128), lambda i: (0, 0)),
        ],
        out_specs=pl.BlockSpec((_ABLK, 128), lambda i: (i, 0)),
        out_shape=jax.ShapeDtypeStruct((_WIDE, 128), jnp.float32),
    )(table_t, table_t, table_t, table_t, W4, b128)


def _gather_extract_sc(tw, uid):
    info = plsc.get_sparse_core_info()
    nc, ns = info.num_cores, info.num_subcores
    nw = nc * ns
    b_per_w = BATCH // nw
    n_chunks = b_per_w // 16
    mesh = plsc.VectorSubcoreMesh(core_axis_name="c", subcore_axis_name="s")

    nslab = 8
    rows_per_slab = b_per_w // nslab
    chunks_per_slab = rows_per_slab // 16

    @functools.partial(
        pl.kernel,
        mesh=mesh,
        out_type=jax.ShapeDtypeStruct((DENSE_OUT, BATCH), jnp.float32),
        scratch_types=[
            pltpu.VMEM((b_per_w,), jnp.int32),
            pltpu.VMEM((b_per_w,), jnp.int32),
            pltpu.VMEM((b_per_w,), jnp.int32),
            pltpu.VMEM((rows_per_slab, 128), jnp.float32),
            pltpu.VMEM((rows_per_slab, 128), jnp.float32),
            pltpu.VMEM((rows_per_slab, 128), jnp.float32),
            pltpu.VMEM((DENSE_OUT, b_per_w), jnp.float32),
            pltpu.SemaphoreType.DMA,
            pltpu.SemaphoreType.DMA,
            pltpu.SemaphoreType.DMA,
            pltpu.SemaphoreType.DMA,
        ],
        compiler_params=pltpu.CompilerParams(needs_layout_passes=False),
    )
    def gather_kernel(tw_hbm, uid_hbm, out_hbm, uid_v, idx_v, lane_v, rows_a,
                      rows_b, rows_c, out_v, sem_a, sem_b, sem_c, sem_o):
        wid = lax.axis_index("s") * nc + lax.axis_index("c")
        base = wid * b_per_w
        pltpu.sync_copy(uid_hbm.at[pl.ds(base, b_per_w)], uid_v)

        def precompute(k, carry):
            u = uid_v[pl.ds(k * 16, 16)]
            idx_v[pl.ds(k * 16, 16)] = lax.bitwise_and(u, _WIDE - 1)
            lane_v[pl.ds(k * 16, 16)] = lax.shift_right_logical(u, 15) * 32
            return carry

        lax.fori_loop(0, n_chunks, precompute, 0)

        bufs = [rows_a, rows_b, rows_c]
        sems = [sem_a, sem_b, sem_c]

        def fire(s, buf, sem):
            return pltpu.async_copy(
                tw_hbm.at[idx_v.at[pl.ds(s * rows_per_slab, rows_per_slab)]],
                buf,
                sem,
            )

        cps = [fire(0, rows_a, sem_a), fire(1, rows_b, sem_b),
               fire(2, rows_c, sem_c)]
        out_cps = []
        iota16 = lax.iota(jnp.int32, 16)

        for s in range(nslab):
            cps[s % 3].wait()
            buf = bufs[s % 3]

            def extract(k, carry, s=s, buf=buf):
                r_local = k * 16
                rows16 = r_local + iota16
                r_glob = s * rows_per_slab + r_local
                lane_base = lane_v[pl.ds(r_glob, 16)]
                for d in range(DENSE_OUT):
                    v = plsc.load_gather(buf, [rows16, lane_base + d])
                    out_v[d, pl.ds(r_glob, 16)] = v
                return carry

            lax.fori_loop(0, chunks_per_slab, extract, 0)
            if s + 3 < nslab:
                cps[s % 3] = fire(s + 3, buf, sems[s % 3])
            if s % 2 == 1:
                c0 = (s - 1) * rows_per_slab
                out_cps.append(
                    pltpu.async_copy(
                        out_v.at[:, pl.ds(c0, 2 * rows_per_slab)],
                        out_hbm.at[:, pl.ds(base + c0, 2 * rows_per_slab)],
                        sem_o,
                    )
                )
        for cp in out_cps:
            cp.wait()

    return gather_kernel(tw, uid)


def kernel(user_id, table, W, b):
    uid = user_id.astype(jnp.int32)
    b128 = jnp.concatenate([b.reshape(1, DENSE_OUT)] * 4, axis=1)
    w4 = jnp.kron(jnp.eye(4, dtype=jnp.float32), W)
    tw = _project_pack_tc(table.T, w4, b128)
    out_t = _gather_extract_sc(tw, uid)
    return out_t.T


# speedup vs baseline: 1.0627x; 1.0627x over previous
import functools

import jax
import jax.numpy as jnp
from jax import lax
from jax.experimental import pallas as pl
from jax.experimental.pallas import tpu as pltpu
from jax.experimental.pallas import tpu_sc as plsc

VOCAB = 100001
EMBED_DIM = 32
DENSE_OUT = 32
BATCH = 16384

_WIDE = 32768
_ABLK = 8192
_AGRID = _WIDE // _ABLK
_LANE_BLOCKS = VOCAB // _ABLK


def _project_pack_tc(table_t, W4, b128):
    def body(t0, t1, t2, t3, w_ref, b_ref, o_ref):
        x4 = jnp.concatenate([t0[...], t1[...], t2[...], t3[...]], axis=0)
        o_ref[...] = (
            lax.dot_general(
                x4,
                w_ref[...],
                (((0,), (0,)), ((), ())),
                preferred_element_type=jnp.float32,
            )
            + b_ref[...]
        )

    def t_spec(j):
        return pl.BlockSpec(
            (EMBED_DIM, _ABLK),
            lambda i: (0, jnp.minimum(i + j * _AGRID, _LANE_BLOCKS)),
        )

    return pl.pallas_call(
        body,
        grid=(_AGRID,),
        in_specs=[
            t_spec(0),
            t_spec(1),
            t_spec(2),
            t_spec(3),
            pl.BlockSpec((128, 128), lambda i: (0, 0)),
            pl.BlockSpec((1, 128), lambda i: (0, 0)),
        ],
        out_specs=pl.BlockSpec((_ABLK, 128), lambda i: (i, 0)),
        out_shape=jax.ShapeDtypeStruct((_WIDE, 128), jnp.float32),
    )(table_t, table_t, table_t, table_t, W4, b128)


def _gather_extract_sc(tw, uid):
    info = plsc.get_sparse_core_info()
    nc, ns = info.num_cores, info.num_subcores
    nw = nc * ns
    b_per_w = BATCH // nw
    n_chunks = b_per_w // 16
    mesh = plsc.VectorSubcoreMesh(core_axis_name="c", subcore_axis_name="s")

    nslab = 4
    rows_per_slab = b_per_w // nslab
    chunks_per_slab = rows_per_slab // 16

    @functools.partial(
        pl.kernel,
        mesh=mesh,
        out_type=jax.ShapeDtypeStruct((DENSE_OUT, BATCH), jnp.float32),
        scratch_types=[
            pltpu.VMEM((b_per_w,), jnp.int32),
            pltpu.VMEM((b_per_w,), jnp.int32),
            pltpu.VMEM((b_per_w,), jnp.int32),
            pltpu.VMEM((rows_per_slab, 128), jnp.float32),
            pltpu.VMEM((rows_per_slab, 128), jnp.float32),
            pltpu.VMEM((DENSE_OUT, b_per_w), jnp.float32),
            pltpu.SemaphoreType.DMA,
            pltpu.SemaphoreType.DMA,
            pltpu.SemaphoreType.DMA,
        ],
        compiler_params=pltpu.CompilerParams(needs_layout_passes=False),
    )
    def gather_kernel(tw_hbm, uid_hbm, out_hbm, uid_v, idx_v, lane_v, rows_a,
                      rows_b, out_v, sem_a, sem_b, sem_o):
        wid = lax.axis_index("s") * nc + lax.axis_index("c")
        base = wid * b_per_w
        pltpu.sync_copy(uid_hbm.at[pl.ds(base, b_per_w)], uid_v)

        def precompute(k, carry):
            u = uid_v[pl.ds(k * 16, 16)]
            idx_v[pl.ds(k * 16, 16)] = lax.bitwise_and(u, _WIDE - 1)
            lane_v[pl.ds(k * 16, 16)] = lax.shift_right_logical(u, 15) * 32
            return carry

        lax.fori_loop(0, n_chunks, precompute, 0)

        bufs = [rows_a, rows_b]
        sems = [sem_a, sem_b]

        def fire(s, buf, sem):
            return pltpu.async_copy(
                tw_hbm.at[idx_v.at[pl.ds(s * rows_per_slab, rows_per_slab)]],
                buf,
                sem,
            )

        cps = [fire(0, rows_a, sem_a), fire(1, rows_b, sem_b)]
        out_cps = []
        iota16 = lax.iota(jnp.int32, 16)

        for s in range(nslab):
            cps[s % 2].wait()
            buf = bufs[s % 2]

            def extract(k, carry, s=s, buf=buf):
                r_local = k * 16
                rows16 = r_local + iota16
                r_glob = s * rows_per_slab + r_local
                lane_base = lane_v[pl.ds(r_glob, 16)]
                for d in range(DENSE_OUT):
                    v = plsc.load_gather(buf, [rows16, lane_base + d])
                    out_v[d, pl.ds(r_glob, 16)] = v
                return carry

            lax.fori_loop(0, chunks_per_slab, extract, 0)
            if s + 2 < nslab:
                cps[s % 2] = fire(s + 2, buf, sems[s % 2])
            out_cps.append(
                pltpu.async_copy(
                    out_v.at[:, pl.ds(s * rows_per_slab, rows_per_slab)],
                    out_hbm.at[
                        :, pl.ds(base + s * rows_per_slab, rows_per_slab)
                    ],
                    sem_o,
                )
            )
        for cp in out_cps:
            cp.wait()

    return gather_kernel(tw, uid)


def kernel(user_id, table, W, b):
    uid = user_id.astype(jnp.int32)
    b128 = jnp.concatenate([b.reshape(1, DENSE_OUT)] * 4, axis=1)
    w4 = jnp.kron(jnp.eye(4, dtype=jnp.float32), W)
    tw = _project_pack_tc(table.T, w4, b128)
    out_t = _gather_extract_sc(tw, uid)
    return out_t.T


# speedup vs baseline: 1.0880x; 1.0238x over previous
import functools

import jax
import jax.numpy as jnp
from jax import lax
from jax.experimental import pallas as pl
from jax.experimental.pallas import tpu as pltpu
from jax.experimental.pallas import tpu_sc as plsc

VOCAB = 100001
EMBED_DIM = 32
DENSE_OUT = 32
BATCH = 16384

_WIDE = 32768
_ABLK = 16384
_AGRID = _WIDE // _ABLK
_LANE_BLOCKS = VOCAB // _ABLK


def _project_pack_tc(table_t, W4, b128):
    def body(t0, t1, t2, t3, w_ref, b_ref, o_ref):
        x4 = jnp.concatenate([t0[...], t1[...], t2[...], t3[...]], axis=0)
        o_ref[...] = (
            lax.dot_general(
                x4,
                w_ref[...],
                (((0,), (0,)), ((), ())),
                preferred_element_type=jnp.float32,
            )
            + b_ref[...]
        )

    def t_spec(j):
        return pl.BlockSpec(
            (EMBED_DIM, _ABLK),
            lambda i: (0, jnp.minimum(i + j * _AGRID, _LANE_BLOCKS)),
        )

    return pl.pallas_call(
        body,
        grid=(_AGRID,),
        in_specs=[
            t_spec(0),
            t_spec(1),
            t_spec(2),
            t_spec(3),
            pl.BlockSpec((128, 128), lambda i: (0, 0)),
            pl.BlockSpec((1, 128), lambda i: (0, 0)),
        ],
        out_specs=pl.BlockSpec((_ABLK, 128), lambda i: (i, 0)),
        out_shape=jax.ShapeDtypeStruct((_WIDE, 128), jnp.float32),
    )(table_t, table_t, table_t, table_t, W4, b128)


def _gather_extract_sc(tw, uid):
    info = plsc.get_sparse_core_info()
    nc, ns = info.num_cores, info.num_subcores
    nw = nc * ns
    b_per_w = BATCH // nw
    n_chunks = b_per_w // 16
    mesh = plsc.VectorSubcoreMesh(core_axis_name="c", subcore_axis_name="s")

    nslab = 4
    rows_per_slab = b_per_w // nslab
    chunks_per_slab = rows_per_slab // 16

    @functools.partial(
        pl.kernel,
        mesh=mesh,
        out_type=jax.ShapeDtypeStruct((DENSE_OUT, BATCH), jnp.float32),
        scratch_types=[
            pltpu.VMEM((b_per_w,), jnp.int32),
            pltpu.VMEM((b_per_w,), jnp.int32),
            pltpu.VMEM((b_per_w,), jnp.int32),
            pltpu.VMEM((rows_per_slab, 128), jnp.float32),
            pltpu.VMEM((rows_per_slab, 128), jnp.float32),
            pltpu.VMEM((DENSE_OUT, b_per_w), jnp.float32),
            pltpu.SemaphoreType.DMA,
            pltpu.SemaphoreType.DMA,
            pltpu.SemaphoreType.DMA,
        ],
        compiler_params=pltpu.CompilerParams(needs_layout_passes=False),
    )
    def gather_kernel(tw_hbm, uid_hbm, out_hbm, uid_v, idx_v, lane_v, rows_a,
                      rows_b, out_v, sem_a, sem_b, sem_o):
        wid = lax.axis_index("s") * nc + lax.axis_index("c")
        base = wid * b_per_w
        pltpu.sync_copy(uid_hbm.at[pl.ds(base, b_per_w)], uid_v)

        def precompute(k, carry):
            u = uid_v[pl.ds(k * 16, 16)]
            idx_v[pl.ds(k * 16, 16)] = lax.bitwise_and(u, _WIDE - 1)
            lane_v[pl.ds(k * 16, 16)] = lax.shift_right_logical(u, 15) * 32
            return carry

        lax.fori_loop(0, n_chunks, precompute, 0)

        bufs = [rows_a, rows_b]
        sems = [sem_a, sem_b]

        def fire(s, buf, sem):
            return pltpu.async_copy(
                tw_hbm.at[idx_v.at[pl.ds(s * rows_per_slab, rows_per_slab)]],
                buf,
                sem,
            )

        cps = [fire(0, rows_a, sem_a), fire(1, rows_b, sem_b)]
        out_cps = []
        iota16 = lax.iota(jnp.int32, 16)

        for s in range(nslab):
            cps[s % 2].wait()
            buf = bufs[s % 2]

            def extract(k, carry, s=s, buf=buf):
                r_local = k * 16
                rows16 = r_local + iota16
                r_glob = s * rows_per_slab + r_local
                lane_base = lane_v[pl.ds(r_glob, 16)]
                for d in range(DENSE_OUT):
                    v = plsc.load_gather(buf, [rows16, lane_base + d])
                    out_v[d, pl.ds(r_glob, 16)] = v
                return carry

            lax.fori_loop(0, chunks_per_slab, extract, 0)
            if s + 2 < nslab:
                cps[s % 2] = fire(s + 2, buf, sems[s % 2])
            out_cps.append(
                pltpu.async_copy(
                    out_v.at[:, pl.ds(s * rows_per_slab, rows_per_slab)],
                    out_hbm.at[
                        :, pl.ds(base + s * rows_per_slab, rows_per_slab)
                    ],
                    sem_o,
                )
            )
        for cp in out_cps:
            cp.wait()

    return gather_kernel(tw, uid)


def kernel(user_id, table, W, b):
    uid = user_id.astype(jnp.int32)
    b128 = jnp.concatenate([b.reshape(1, DENSE_OUT)] * 4, axis=1)
    w4 = jnp.kron(jnp.eye(4, dtype=jnp.float32), W)
    tw = _project_pack_tc(table.T, w4, b128)
    out_t = _gather_extract_sc(tw, uid)
    return out_t.T
